# Initial kernel scaffold; baseline (speedup 1.0000x reference)
import jax
import jax.numpy as jnp
from jax.experimental import pallas as pl


def kernel(x, W1, b1, W2, b2, src, rel, dst):
    raise NotImplementedError("write your pallas kernel here")


# speedup vs baseline: 11.2740x; 11.2740x over previous
import functools

import jax
import jax.numpy as jnp
from jax import lax
from jax.experimental import pallas as pl
from jax.experimental.pallas import tpu as pltpu
from jax.experimental.pallas import tpu_sc as plsc

N = 10000
NREL = 8
R = 2 * NREL + 1
NFEAT = 128
NHID = 16
E = 160000

TBL = 81920
DUM = 80000
CH = 128
NBUF = 8
EPT = 10240
NCHUNK = EPT // CH
NGROUP = NCHUNK // NBUF
EPAD = 16 * EPT
RPT = TBL // 16

_BLK = 1000
_GRID = N // _BLK


def _tc1_body(x_ref, w_ref, o_ref, oself_ref):
    t = jnp.dot(x_ref[...], w_ref[...], preferred_element_type=jnp.float32)
    o_ref[...] = t
    oself_ref[...] = t[:, 2 * NREL * NHID:]


def _tc1(x, w1cat):
    return pl.pallas_call(
        _tc1_body,
        grid=(_GRID,),
        in_specs=[
            pl.BlockSpec((_BLK, NFEAT), lambda i: (i, 0)),
            pl.BlockSpec((NFEAT, R * NHID), lambda i: (0, 0)),
        ],
        out_specs=[
            pl.BlockSpec((_BLK, R * NHID), lambda i: (i, 0)),
            pl.BlockSpec((_BLK, NHID), lambda i: (i, 0)),
        ],
        out_shape=[
            jax.ShapeDtypeStruct((N, R * NHID), jnp.float32),
            jax.ShapeDtypeStruct((N, NHID), jnp.float32),
        ],
    )(x, w1cat)


def _tc2_body(t1s_ref, a_ref, cnt_ref, w2_ref, b1_ref,
              t2_ref, t2s_ref, recip_ref):
    cnt = cnt_ref[...]
    rec = jnp.where(cnt > 0, 1.0 / jnp.maximum(cnt, 1.0), 0.0)
    recip_ref[...] = rec
    msg = jnp.sum(a_ref[...] * rec[..., None], axis=(0, 2))
    h = jnp.maximum(t1s_ref[...] + msg + b1_ref[...], 0.0)
    t2 = jnp.dot(h, w2_ref[...], preferred_element_type=jnp.float32)
    t2_ref[...] = t2
    t2s_ref[...] = t2[:, 2 * NREL * NHID:]


def _tc2(t1self, a, cnt, w2cat, b1):
    a4 = a.reshape(2, TBL // NREL, NREL, NHID)
    cnt3 = cnt.reshape(2, TBL // NREL, NREL)
    return pl.pallas_call(
        _tc2_body,
        grid=(_GRID,),
        in_specs=[
            pl.BlockSpec((_BLK, NHID), lambda i: (i, 0)),
            pl.BlockSpec((2, _BLK, NREL, NHID), lambda i: (0, i, 0, 0)),
            pl.BlockSpec((2, _BLK, NREL), lambda i: (0, i, 0)),
            pl.BlockSpec((NHID, R * NHID), lambda i: (0, 0)),
            pl.BlockSpec((1, NHID), lambda i: (0, 0)),
        ],
        out_specs=[
            pl.BlockSpec((_BLK, R * NHID), lambda i: (i, 0)),
            pl.BlockSpec((_BLK, NHID), lambda i: (i, 0)),
            pl.BlockSpec((2, _BLK, NREL), lambda i: (0, i, 0)),
        ],
        out_shape=[
            jax.ShapeDtypeStruct((N, R * NHID), jnp.float32),
            jax.ShapeDtypeStruct((N, NHID), jnp.float32),
            jax.ShapeDtypeStruct((2, TBL // NREL, NREL), jnp.float32),
        ],
    )(t1self, a4, cnt3, w2cat, b1)


def _tc3_body(t2s_ref, b_ref, recip_ref, b2_ref, out_ref):
    msg = jnp.sum(b_ref[...] * recip_ref[...][..., None], axis=(0, 2))
    o = t2s_ref[...] + msg + b2_ref[...]
    m = jnp.max(o, axis=1, keepdims=True)
    ex = jnp.exp(o - m)
    out_ref[...] = (o - m) - jnp.log(jnp.sum(ex, axis=1, keepdims=True))


def _tc3(t2self, b, recip, b2):
    b4 = b.reshape(2, TBL // NREL, NREL, NHID)
    return pl.pallas_call(
        _tc3_body,
        grid=(_GRID,),
        in_specs=[
            pl.BlockSpec((_BLK, NHID), lambda i: (i, 0)),
            pl.BlockSpec((2, _BLK, NREL, NHID), lambda i: (0, i, 0, 0)),
            pl.BlockSpec((2, _BLK, NREL), lambda i: (0, i, 0)),
            pl.BlockSpec((1, NHID), lambda i: (0, 0)),
        ],
        out_specs=pl.BlockSpec((_BLK, NHID), lambda i: (i, 0)),
        out_shape=jax.ShapeDtypeStruct((N, NHID), jnp.float32),
    )(t2self, b4, recip, b2)


def _sc_mesh():
    return plsc.VectorSubcoreMesh(core_axis_name="c", subcore_axis_name="s")


def _sc_body(with_cnt, *refs):
    if with_cnt:
        (tbl, gidx, sidx, a_out, cnt_out,
         gbuf, sbuf, rows, onesb, zbuf, zrow, acc, cacc,
         sem_g, sem_s, sem_c) = refs
    else:
        (tbl, gidx, sidx, a_out,
         gbuf, sbuf, rows, zbuf, acc,
         sem_g, sem_s) = refs
    cc = lax.axis_index("c")
    ss = lax.axis_index("s")
    base_row = ss * RPT

    pltpu.sync_copy(gidx.at[cc, ss], gbuf)
    pltpu.sync_copy(sidx.at[cc, ss], sbuf)

    zero16 = jnp.zeros((16,), jnp.float32)
    for i in range(CH):
        zbuf[i] = zero16
    if with_cnt:
        one16 = jnp.ones((16,), jnp.float32)
        for i in range(CH // 16):
            onesb[pl.ds(i * 16, 16)] = one16
            zrow[pl.ds(i * 16, 16)] = zero16

    for j in range(RPT // CH):
        pltpu.sync_copy(zbuf, acc.at[pl.ds(base_row + j * CH, CH)])
        if with_cnt:
            pltpu.sync_copy(zrow, cacc.at[pl.ds(base_row + j * CH, CH)])
    plsc.subcore_barrier()

    def group(gi, carry):
        k0 = gi * NBUF
        ghs = [pltpu.async_copy(tbl.at[gbuf.at[k0 + b]], rows.at[b], sem_g)
               for b in range(NBUF)]
        shs = []
        for b in range(NBUF):
            ghs[b].wait()
            shs.append(pltpu.async_copy(rows.at[b], acc.at[sbuf.at[k0 + b]],
                                        sem_s, add=True))
            if with_cnt:
                shs.append(pltpu.async_copy(onesb, cacc.at[sbuf.at[k0 + b]],
                                            sem_c, add=True))
        for h in shs:
            h.wait()
        return carry

    lax.fori_loop(0, NGROUP, group, 0)
    plsc.subcore_barrier()

    pltpu.sync_copy(acc.at[pl.ds(base_row, RPT)],
                    a_out.at[cc, pl.ds(base_row, RPT)])
    if with_cnt:
        pltpu.sync_copy(cacc.at[pl.ds(base_row, RPT)],
                        cnt_out.at[cc, pl.ds(base_row, RPT)])


def _sc_scatter(tblflat, gidx, sidx, with_cnt):
    if with_cnt:
        out_type = [jax.ShapeDtypeStruct((2, TBL, NHID), jnp.float32),
                    jax.ShapeDtypeStruct((2, TBL), jnp.float32)]
    else:
        out_type = jax.ShapeDtypeStruct((2, TBL, NHID), jnp.float32)
    scratch = [
        pltpu.VMEM((NCHUNK, CH), jnp.int32),
        pltpu.VMEM((NCHUNK, CH), jnp.int32),
        pltpu.VMEM((NBUF, CH, NHID), jnp.float32),
    ]
    if with_cnt:
        scratch += [pltpu.VMEM((CH,), jnp.float32)]
    scratch += [pltpu.VMEM((CH, NHID), jnp.float32)]
    if with_cnt:
        scratch += [pltpu.VMEM((CH,), jnp.float32)]
    scratch += [pltpu.VMEM_SHARED((TBL, NHID), jnp.float32)]
    if with_cnt:
        scratch += [pltpu.VMEM_SHARED((TBL,), jnp.float32)]
    scratch += [pltpu.SemaphoreType.DMA, pltpu.SemaphoreType.DMA]
    if with_cnt:
        scratch += [pltpu.SemaphoreType.DMA]
    return pl.kernel(
        functools.partial(_sc_body, with_cnt),
        out_type=out_type,
        mesh=_sc_mesh(),
        scratch_types=scratch,
        compiler_params=pltpu.CompilerParams(use_tc_tiling_on_sc=False),
    )(tblflat, gidx, sidx)


def kernel(x, W1, b1, W2, b2, src, rel, dst):
    w1cat = W1.transpose(1, 0, 2).reshape(NFEAT, R * NHID)
    w2cat = W2.transpose(1, 0, 2).reshape(NHID, R * NHID)

    g0 = dst * R + rel
    k0 = src * NREL + rel
    g1 = src * R + (rel + NREL)
    k1 = dst * NREL + rel
    pad = EPAD - E
    gpad = jnp.zeros((pad,), jnp.int32)
    spad = jnp.full((pad,), DUM, jnp.int32)
    gidx = jnp.stack([jnp.concatenate([g0, gpad]),
                      jnp.concatenate([g1, gpad])]).reshape(2, 16, NCHUNK, CH)
    sidx = jnp.stack([jnp.concatenate([k0, spad]),
                      jnp.concatenate([k1, spad])]).reshape(2, 16, NCHUNK, CH)

    t1, t1self = _tc1(x, w1cat)
    a, cnt = _sc_scatter(t1.reshape(N * R, NHID), gidx, sidx, True)
    t2, t2self, recip = _tc2(t1self, a, cnt, w2cat, b1.reshape(1, NHID))
    b = _sc_scatter(t2.reshape(N * R, NHID), gidx, sidx, False)
    return _tc3(t2self, b, recip, b2.reshape(1, NHID))
